# Initial kernel scaffold; baseline (speedup 1.0000x reference)
import jax
import jax.numpy as jnp
from jax.experimental import pallas as pl


def kernel(params, x, edge_index):
    raise NotImplementedError("write your pallas kernel here")


# speedup vs baseline: 4.9340x; 4.9340x over previous
import functools

import jax
import jax.numpy as jnp
from jax import lax
from jax.experimental import pallas as pl
from jax.experimental.pallas import tpu as pltpu
from jax.experimental.pallas import tpu_sc as plsc

_VOCAB = 64
_IN = 64
_HID = 13
_N = 1024
_SEQ = 16
_E = 16384
_G = 4 * _HID

_HP = lax.Precision.HIGHEST


def _dot(a, b):
    return jnp.dot(a, b, precision=_HP, preferred_element_type=jnp.float32)


def _cell(g, c):
    i = jax.nn.sigmoid(g[:, 0:_HID])
    f = jax.nn.sigmoid(g[:, _HID:2 * _HID])
    gg = jnp.tanh(g[:, 2 * _HID:3 * _HID])
    o = jax.nn.sigmoid(g[:, 3 * _HID:4 * _HID])
    c2 = f * c + i * gg
    return o * jnp.tanh(c2), c2


def _cell2(g, c):
    def pick(j):
        return jnp.concatenate(
            [g[:, j * _HID:(j + 1) * _HID],
             g[:, _G + j * _HID:_G + (j + 1) * _HID]], axis=1)
    i = jax.nn.sigmoid(pick(0))
    f = jax.nn.sigmoid(pick(1))
    gg = jnp.tanh(pick(2))
    o = jax.nn.sigmoid(pick(3))
    c2 = f * c + i * gg
    return o * jnp.tanh(c2), c2


def _leaky(v):
    return jnp.where(v >= 0, v, 0.01 * v)


def _sc_counts(edge_index):
    info = plsc.get_sparse_core_info()
    nc, ns = info.num_cores, info.num_subcores
    ntiles = nc * ns
    e_per = _E // ntiles
    chunk = 128
    nchunk = e_per // chunk
    words_per_tile = (_N * _N) // ns
    zb = 2048

    mesh = plsc.VectorSubcoreMesh(core_axis_name="c", subcore_axis_name="s")

    @functools.partial(
        pl.kernel,
        out_type=jax.ShapeDtypeStruct((nc, _N * _N), jnp.float32),
        mesh=mesh,
        scratch_types=[
            pltpu.VMEM((e_per,), jnp.int32),
            pltpu.VMEM((e_per,), jnp.int32),
            pltpu.VMEM((nchunk, chunk), jnp.int32),
            pltpu.VMEM((chunk,), jnp.float32),
            pltpu.VMEM((zb,), jnp.float32),
            pltpu.VMEM_SHARED((_N * _N,), jnp.float32),
        ],
    )
    def counts_kernel(edges, out, src_v, dst_v, idx_v, ones_v, zb_v, acc):
        cid = lax.axis_index("c")
        sid = lax.axis_index("s")
        gid = cid * ns + sid

        zeros16 = jnp.zeros((16,), jnp.float32)
        ones16 = jnp.ones((16,), jnp.float32)
        for k in range(zb // 16):
            zb_v[pl.ds(k * 16, 16)] = zeros16
        for k in range(chunk // 16):
            ones_v[pl.ds(k * 16, 16)] = ones16

        base = gid * e_per
        pltpu.sync_copy(edges.at[0, pl.ds(base, e_per)], src_v)
        pltpu.sync_copy(edges.at[1, pl.ds(base, e_per)], dst_v)
        for j in range(nchunk):
            for k in range(chunk // 16):
                o = j * chunk + k * 16
                s16 = src_v[pl.ds(o, 16)]
                d16 = dst_v[pl.ds(o, 16)]
                idx_v[j, pl.ds(k * 16, 16)] = d16 * _N + s16

        zbase = sid * words_per_tile
        for j in range(words_per_tile // zb):
            pltpu.sync_copy(zb_v, acc.at[pl.ds(zbase + j * zb, zb)])
        plsc.subcore_barrier()

        for j in range(nchunk):
            pltpu.sync_copy(ones_v, acc.at[idx_v.at[j]], add=True)
        plsc.subcore_barrier()

        pltpu.sync_copy(acc.at[pl.ds(zbase, words_per_tile)],
                        out.at[cid, pl.ds(zbase, words_per_tile)])

    return counts_kernel(edge_index)


def _enc_body(xT_ref, emb_ref, wih0_ref, whh0b_ref, b0_ref,
              wih1_ref, whh1b_ref, b1_ref,
              p1w_ref, p1b_ref, p2w_ref, p2b_ref,
              dwih_ref, db_ref, ow_ref, ob_ref,
              tail_ref, sh_ref, sc_ref, hs_ref):
    emb = emb_ref[...]
    iota = lax.broadcasted_iota(jnp.int32, (_N, _VOCAB), 1)

    def oh(t):
        return (xT_ref[t] == iota).astype(jnp.float32)

    ewf = _dot(emb, wih0_ref[0])
    ewb = _dot(emb, wih0_ref[1])
    whh0b = whh0b_ref[...]
    b0 = b0_ref[...]
    z = jnp.zeros((_N, 2 * _HID), jnp.float32)

    def step0(s, hc):
        h, c = hc
        ihf = _dot(oh(s), ewf)
        ihb = _dot(oh(_SEQ - 1 - s), ewb)
        g = jnp.concatenate([ihf, ihb], axis=1) + b0 + _dot(h, whh0b)
        h, c = _cell2(g, c)
        hs_ref[s] = h
        return (h, c)

    h0T, c0T = lax.fori_loop(0, _SEQ, step0, (z, z))

    whh1b = whh1b_ref[...]
    b1 = b1_ref[...]

    def step1(s, hc):
        h, c = hc
        a = hs_ref[s]
        bw = hs_ref[_SEQ - 1 - s]
        in_f = jnp.concatenate([a[:, :_HID], bw[:, _HID:]], axis=1)
        in_b = jnp.concatenate([bw[:, :_HID], a[:, _HID:]], axis=1)
        g = jnp.concatenate([_dot(in_f, wih1_ref[0]),
                             _dot(in_b, wih1_ref[1])], axis=1)
        g = g + b1 + _dot(h, whh1b)
        return _cell2(g, c)

    h1T, c1T = lax.fori_loop(0, _SEQ, step1, (z, z))

    h_cat = jnp.concatenate([h0T, h1T], axis=1)
    c_cat = jnp.concatenate([c0T, c1T], axis=1)
    sh_ref[...] = _dot(h_cat, p1w_ref[...]) + p1b_ref[...]
    sc_ref[...] = _dot(c_cat, p2w_ref[...]) + p2b_ref[...]

    vs = lax.broadcasted_iota(jnp.int32, (_VOCAB, 1), 0).astype(jnp.float32)
    zc = jnp.zeros((_VOCAB, _HID), jnp.float32)
    h2s = []
    for d in range(2):
        g = vs * dwih_ref[d] + db_ref[d]
        h2, _ = _cell(g, zc)
        h2s.append(h2)
    table = _dot(jnp.concatenate(h2s, axis=1), ow_ref[...]) + ob_ref[...]

    def stept(t, _):
        tail_ref[t] = _dot(oh(t), table)
        return _

    lax.fori_loop(0, _SEQ - 1, stept, 0)


def _gnn_dec_body(c_ref, sh_ref, sc_ref,
                  hw1_ref, hb1_ref, hw2_ref, hb2_ref, hfw_ref, hfb_ref,
                  cw1_ref, cb1_ref, cw2_ref, cb2_ref, cfw_ref, cfb_ref,
                  dwih_ref, dwhh_ref, db_ref, ow_ref, ob_ref,
                  head_ref):
    C = c_ref[0] + c_ref[1]
    deg = jnp.sum(C, axis=1, keepdims=True) + 1.0
    dinv = lax.rsqrt(deg)
    d2 = dinv * dinv

    def conv(h, w_ref, b_ref):
        hw = _dot(h, w_ref[...])
        return dinv * _dot(C, dinv * hw) + d2 * hw + b_ref[...]

    def gnn(s, w1, b1, w2, b2, fw, fb):
        h1 = _leaky(conv(s, w1, b1))
        h2 = _leaky(conv(h1, w2, b2))
        return _dot(h2, fw[...]) + fb[...]

    shg = gnn(sh_ref[...], hw1_ref, hb1_ref, hw2_ref, hb2_ref, hfw_ref, hfb_ref)
    scg = gnn(sc_ref[...], cw1_ref, cb1_ref, cw2_ref, cb2_ref, cfw_ref, cfb_ref)

    h2s = []
    for d in range(2):
        hx = shg[:, d * _HID:(d + 1) * _HID]
        cx = scg[:, d * _HID:(d + 1) * _HID]
        g = -dwih_ref[d] + _dot(hx, dwhh_ref[d]) + db_ref[d]
        h2, _ = _cell(g, cx)
        h2s.append(h2)
    head_ref[...] = _dot(jnp.concatenate(h2s, axis=1), ow_ref[...]) + ob_ref[...]


def _prep(params):
    enc = params['enc']

    def pack(l):
        wih = jnp.stack([enc[l][d]['Wih'].T for d in range(2)])
        whhb = jnp.zeros((2 * _HID, 2 * _G), jnp.float32)
        whhb = whhb.at[:_HID, :_G].set(enc[l][0]['Whh'].T)
        whhb = whhb.at[_HID:, _G:].set(enc[l][1]['Whh'].T)
        b = jnp.concatenate(
            [(enc[l][d]['bih'] + enc[l][d]['bhh']) for d in range(2)])[None, :]
        return wih, whhb, b

    wih0, whh0b, b0 = pack(0)
    wih1, whh1b, b1 = pack(1)
    dec = params['dec']
    dwih = jnp.stack([dec[k]['Wih'].T for k in ('f', 'b')])
    dwhh = jnp.stack([dec[k]['Whh'].T for k in ('f', 'b')])
    db = jnp.stack([(dec[k]['bih'] + dec[k]['bhh'])[None, :]
                    for k in ('f', 'b')])
    return dict(
        emb=params['emb'][:_VOCAB],
        wih0=wih0, whh0b=whh0b, b0=b0, wih1=wih1, whh1b=whh1b, b1=b1,
        p1w=params['proj1_w'], p1b=params['proj1_b'][None, :],
        p2w=params['proj2_w'], p2b=params['proj2_b'][None, :],
        dwih=dwih, dwhh=dwhh, db=db,
        ow=params['out_w'], ob=params['out_b'][None, :],
    )


def _gp(g):
    return (g['w1'], g['b1'][None, :], g['w2'], g['b2'][None, :],
            g['fc_w'], g['fc_b'][None, :])


def kernel(params, x, edge_index):
    p = _prep(params)
    counts = _sc_counts(edge_index).reshape(2, _N, _N)

    xT = x.T[:, :, None]
    tail, sh, sc = pl.pallas_call(
        _enc_body,
        out_shape=[
            jax.ShapeDtypeStruct((_SEQ - 1, _N, _VOCAB), jnp.float32),
            jax.ShapeDtypeStruct((_N, 2 * _HID), jnp.float32),
            jax.ShapeDtypeStruct((_N, 2 * _HID), jnp.float32),
        ],
        scratch_shapes=[pltpu.VMEM((_SEQ, _N, 2 * _HID), jnp.float32)],
    )(xT, p['emb'], p['wih0'], p['whh0b'], p['b0'],
      p['wih1'], p['whh1b'], p['b1'],
      p['p1w'], p['p1b'], p['p2w'], p['p2b'],
      p['dwih'], p['db'], p['ow'], p['ob'])

    head = pl.pallas_call(
        _gnn_dec_body,
        out_shape=jax.ShapeDtypeStruct((_N, _VOCAB), jnp.float32),
    )(counts, sh, sc, *_gp(params['gnn_h']), *_gp(params['gnn_c']),
      p['dwih'], p['dwhh'], p['db'], p['ow'], p['ob'])

    return jnp.concatenate([head[:, None, :], tail.transpose(1, 0, 2)], axis=1)


# speedup vs baseline: 6.9732x; 1.4133x over previous
import functools

import jax
import jax.numpy as jnp
from jax import lax
from jax.experimental import pallas as pl
from jax.experimental.pallas import tpu as pltpu
from jax.experimental.pallas import tpu_sc as plsc

_VOCAB = 64
_IN = 64
_HID = 13
_N = 1024
_SEQ = 16
_E = 16384
_G = 4 * _HID

_HP = lax.Precision.DEFAULT


def _dot(a, b):
    return jnp.dot(a, b, precision=_HP, preferred_element_type=jnp.float32)


def _cell(g, c):
    i = jax.nn.sigmoid(g[:, 0:_HID])
    f = jax.nn.sigmoid(g[:, _HID:2 * _HID])
    gg = jnp.tanh(g[:, 2 * _HID:3 * _HID])
    o = jax.nn.sigmoid(g[:, 3 * _HID:4 * _HID])
    c2 = f * c + i * gg
    return o * jnp.tanh(c2), c2


def _cell2(g, c):
    def pick(j):
        return jnp.concatenate(
            [g[:, j * _HID:(j + 1) * _HID],
             g[:, _G + j * _HID:_G + (j + 1) * _HID]], axis=1)
    i = jax.nn.sigmoid(pick(0))
    f = jax.nn.sigmoid(pick(1))
    gg = jnp.tanh(pick(2))
    o = jax.nn.sigmoid(pick(3))
    c2 = f * c + i * gg
    return o * jnp.tanh(c2), c2


def _leaky(v):
    return jnp.where(v >= 0, v, 0.01 * v)


def _sc_counts(edge_index):
    info = plsc.get_sparse_core_info()
    nc, ns = info.num_cores, info.num_subcores
    ntiles = nc * ns
    e_per = _E // ntiles
    chunk = 128
    nchunk = e_per // chunk
    words_per_tile = (_N * _N) // ns
    zb = 2048

    mesh = plsc.VectorSubcoreMesh(core_axis_name="c", subcore_axis_name="s")

    @functools.partial(
        pl.kernel,
        out_type=jax.ShapeDtypeStruct((nc, _N * _N), jnp.float32),
        mesh=mesh,
        scratch_types=[
            pltpu.VMEM((e_per,), jnp.int32),
            pltpu.VMEM((e_per,), jnp.int32),
            pltpu.VMEM((nchunk, chunk), jnp.int32),
            pltpu.VMEM((chunk,), jnp.float32),
            pltpu.VMEM((zb,), jnp.float32),
            pltpu.VMEM_SHARED((_N * _N,), jnp.float32),
        ],
    )
    def counts_kernel(edges, out, src_v, dst_v, idx_v, ones_v, zb_v, acc):
        cid = lax.axis_index("c")
        sid = lax.axis_index("s")
        gid = cid * ns + sid

        zeros16 = jnp.zeros((16,), jnp.float32)
        ones16 = jnp.ones((16,), jnp.float32)
        for k in range(zb // 16):
            zb_v[pl.ds(k * 16, 16)] = zeros16
        for k in range(chunk // 16):
            ones_v[pl.ds(k * 16, 16)] = ones16

        base = gid * e_per
        pltpu.sync_copy(edges.at[0, pl.ds(base, e_per)], src_v)
        pltpu.sync_copy(edges.at[1, pl.ds(base, e_per)], dst_v)
        for j in range(nchunk):
            for k in range(chunk // 16):
                o = j * chunk + k * 16
                s16 = src_v[pl.ds(o, 16)]
                d16 = dst_v[pl.ds(o, 16)]
                idx_v[j, pl.ds(k * 16, 16)] = d16 * _N + s16

        zbase = sid * words_per_tile
        for j in range(words_per_tile // zb):
            pltpu.sync_copy(zb_v, acc.at[pl.ds(zbase + j * zb, zb)])
        plsc.subcore_barrier()

        for j in range(nchunk):
            pltpu.sync_copy(ones_v, acc.at[idx_v.at[j]], add=True)
        plsc.subcore_barrier()

        pltpu.sync_copy(acc.at[pl.ds(zbase, words_per_tile)],
                        out.at[cid, pl.ds(zbase, words_per_tile)])

    return counts_kernel(edge_index)


def _enc_body(xT_ref, emb_ref, wih0_ref, whh0b_ref, b0_ref,
              wih1_ref, whh1b_ref, b1_ref,
              p1w_ref, p1b_ref, p2w_ref, p2b_ref,
              dwih_ref, db_ref, ow_ref, ob_ref,
              tail_ref, sh_ref, sc_ref, hs_ref):
    emb = emb_ref[...]
    iota = lax.broadcasted_iota(jnp.int32, (_N, _VOCAB), 1)

    def oh(t):
        return (xT_ref[t] == iota).astype(jnp.float32)

    ewf = _dot(emb, wih0_ref[0])
    ewb = _dot(emb, wih0_ref[1])
    whh0b = whh0b_ref[...]
    b0 = b0_ref[...]
    z = jnp.zeros((_N, 2 * _HID), jnp.float32)

    def step0(s, hc):
        h, c = hc
        ihf = _dot(oh(s), ewf)
        ihb = _dot(oh(_SEQ - 1 - s), ewb)
        g = jnp.concatenate([ihf, ihb], axis=1) + b0 + _dot(h, whh0b)
        h, c = _cell2(g, c)
        hs_ref[s] = h
        return (h, c)

    h0T, c0T = lax.fori_loop(0, _SEQ, step0, (z, z))

    whh1b = whh1b_ref[...]
    b1 = b1_ref[...]

    def step1(s, hc):
        h, c = hc
        a = hs_ref[s]
        bw = hs_ref[_SEQ - 1 - s]
        in_f = jnp.concatenate([a[:, :_HID], bw[:, _HID:]], axis=1)
        in_b = jnp.concatenate([bw[:, :_HID], a[:, _HID:]], axis=1)
        g = jnp.concatenate([_dot(in_f, wih1_ref[0]),
                             _dot(in_b, wih1_ref[1])], axis=1)
        g = g + b1 + _dot(h, whh1b)
        return _cell2(g, c)

    h1T, c1T = lax.fori_loop(0, _SEQ, step1, (z, z))

    h_cat = jnp.concatenate([h0T, h1T], axis=1)
    c_cat = jnp.concatenate([c0T, c1T], axis=1)
    sh_ref[...] = _dot(h_cat, p1w_ref[...]) + p1b_ref[...]
    sc_ref[...] = _dot(c_cat, p2w_ref[...]) + p2b_ref[...]

    vs = lax.broadcasted_iota(jnp.int32, (_VOCAB, 1), 0).astype(jnp.float32)
    zc = jnp.zeros((_VOCAB, _HID), jnp.float32)
    h2s = []
    for d in range(2):
        g = vs * dwih_ref[d] + db_ref[d]
        h2, _ = _cell(g, zc)
        h2s.append(h2)
    table = _dot(jnp.concatenate(h2s, axis=1), ow_ref[...]) + ob_ref[...]
    for t in range(_SEQ - 1):
        tail_ref[:, t + 1, :] = _dot(oh(t), table)


def _gnn_dec_body(c_ref, sh_ref, sc_ref,
                  hw1_ref, hb1_ref, hw2_ref, hb2_ref, hfw_ref, hfb_ref,
                  cw1_ref, cb1_ref, cw2_ref, cb2_ref, cfw_ref, cfb_ref,
                  dwih_ref, dwhh_ref, db_ref, ow_ref, ob_ref,
                  head_ref):
    C = c_ref[0] + c_ref[1]
    deg = jnp.sum(C, axis=1, keepdims=True) + 1.0
    dinv = lax.rsqrt(deg)
    d2 = dinv * dinv

    def conv2(ha, hc, wa_ref, ba_ref, wc_ref, bc_ref):
        wa = wa_ref[...]
        hw = jnp.concatenate([_dot(ha, wa), _dot(hc, wc_ref[...])], axis=1)
        agg = dinv * _dot(C, dinv * hw) + d2 * hw
        k = wa.shape[1]
        return agg[:, :k] + ba_ref[...], agg[:, k:] + bc_ref[...]

    a1, c1 = conv2(sh_ref[...], sc_ref[...], hw1_ref, hb1_ref, cw1_ref, cb1_ref)
    a2, c2 = conv2(_leaky(a1), _leaky(c1), hw2_ref, hb2_ref, cw2_ref, cb2_ref)
    shg = _dot(_leaky(a2), hfw_ref[...]) + hfb_ref[...]
    scg = _dot(_leaky(c2), cfw_ref[...]) + cfb_ref[...]

    h2s = []
    for d in range(2):
        hx = shg[:, d * _HID:(d + 1) * _HID]
        cx = scg[:, d * _HID:(d + 1) * _HID]
        g = -dwih_ref[d] + _dot(hx, dwhh_ref[d]) + db_ref[d]
        h2, _ = _cell(g, cx)
        h2s.append(h2)
    head_ref[...] = _dot(jnp.concatenate(h2s, axis=1), ow_ref[...]) + ob_ref[...]


def _prep(params):
    enc = params['enc']

    def pack(l):
        wih = jnp.stack([enc[l][d]['Wih'].T for d in range(2)])
        whhb = jnp.zeros((2 * _HID, 2 * _G), jnp.float32)
        whhb = whhb.at[:_HID, :_G].set(enc[l][0]['Whh'].T)
        whhb = whhb.at[_HID:, _G:].set(enc[l][1]['Whh'].T)
        b = jnp.concatenate(
            [(enc[l][d]['bih'] + enc[l][d]['bhh']) for d in range(2)])[None, :]
        return wih, whhb, b

    wih0, whh0b, b0 = pack(0)
    wih1, whh1b, b1 = pack(1)
    dec = params['dec']
    dwih = jnp.stack([dec[k]['Wih'].T for k in ('f', 'b')])
    dwhh = jnp.stack([dec[k]['Whh'].T for k in ('f', 'b')])
    db = jnp.stack([(dec[k]['bih'] + dec[k]['bhh'])[None, :]
                    for k in ('f', 'b')])
    return dict(
        emb=params['emb'][:_VOCAB],
        wih0=wih0, whh0b=whh0b, b0=b0, wih1=wih1, whh1b=whh1b, b1=b1,
        p1w=params['proj1_w'], p1b=params['proj1_b'][None, :],
        p2w=params['proj2_w'], p2b=params['proj2_b'][None, :],
        dwih=dwih, dwhh=dwhh, db=db,
        ow=params['out_w'], ob=params['out_b'][None, :],
    )


def _gp(g):
    return (g['w1'], g['b1'][None, :], g['w2'], g['b2'][None, :],
            g['fc_w'], g['fc_b'][None, :])


def kernel(params, x, edge_index):
    p = _prep(params)
    counts = _sc_counts(edge_index).reshape(2, _N, _N)

    xT = x.T[:, :, None]
    tail, sh, sc = pl.pallas_call(
        _enc_body,
        out_shape=[
            jax.ShapeDtypeStruct((_N, _SEQ, _VOCAB), jnp.float32),
            jax.ShapeDtypeStruct((_N, 2 * _HID), jnp.float32),
            jax.ShapeDtypeStruct((_N, 2 * _HID), jnp.float32),
        ],
        scratch_shapes=[pltpu.VMEM((_SEQ, _N, 2 * _HID), jnp.float32)],
    )(xT, p['emb'], p['wih0'], p['whh0b'], p['b0'],
      p['wih1'], p['whh1b'], p['b1'],
      p['p1w'], p['p1b'], p['p2w'], p['p2b'],
      p['dwih'], p['db'], p['ow'], p['ob'])

    head = pl.pallas_call(
        _gnn_dec_body,
        out_shape=jax.ShapeDtypeStruct((_N, _VOCAB), jnp.float32),
    )(counts, sh, sc, *_gp(params['gnn_h']), *_gp(params['gnn_c']),
      p['dwih'], p['dwhh'], p['db'], p['ow'], p['ob'])

    return tail.at[:, 0, :].set(head)
